# baseline (speedup 1.0000x reference)
import functools

import jax
import jax.numpy as jnp
from jax import lax
from jax.experimental import pallas as pl
from jax.experimental.pallas import tpu as pltpu
from jax.experimental.pallas import tpu_sc as plsc

B = 1024
S = 1024
D = 64
NC, NS = 2, 16
NW = NC * NS
BPW = B // NW
SCH = 128
NST = S // SCH
NCHUNK = BPW * NST
NBUF = 4

_mesh = plsc.VectorSubcoreMesh(
    core_axis_name="c", subcore_axis_name="s", num_cores=NC, num_subcores=NS
)


@functools.partial(
    pl.kernel,
    out_type=jax.ShapeDtypeStruct((B, D // 8, S // 128, 8, 128), jnp.float32),
    mesh=_mesh,
    scratch_types=(
        [pltpu.VMEM((SCH,), jnp.int32) for _ in range(NBUF)]
        + [pltpu.VMEM((SCH, D), jnp.float32) for _ in range(NBUF)]
        + [pltpu.VMEM((D, SCH + 1), jnp.float32) for _ in range(NBUF)]
        + [pltpu.SemaphoreType.DMA for _ in range(3 * NBUF)]
    ),
    compiler_params=pltpu.CompilerParams(
        use_tc_tiling_on_sc=False,
        needs_layout_passes=False,
        disable_bounds_checks=True,
    ),
)
def _embed(tv_hbm, table_hbm, pos_hbm, out_hbm, *scratch):
    idx = scratch[:NBUF]
    src = scratch[NBUF:2 * NBUF]
    obuf = scratch[2 * NBUF:3 * NBUF]
    sip = scratch[3 * NBUF:4 * NBUF]
    sg = scratch[4 * NBUF:5 * NBUF]
    so = scratch[5 * NBUF:6 * NBUF]
    wid = lax.axis_index("s") * NC + lax.axis_index("c")
    b0 = wid * BPW

    iota = lax.iota(jnp.int32, 16)
    dv = [iota + 16 * j for j in range(D // 16)]

    def start(g, b):
        bidx = b0 + g // NST
        s0 = lax.rem(g, NST) * SCH
        pltpu.async_copy(tv_hbm.at[bidx, pl.ds(s0, SCH)], idx[b], sip[b])
        pltpu.async_copy(pos_hbm.at[pl.ds(s0, SCH)], src[b], sip[b])

    def wait_ip(b):
        pltpu.make_async_copy(tv_hbm.at[0, pl.ds(0, SCH)], idx[b], sip[b]).wait()
        pltpu.make_async_copy(pos_hbm.at[pl.ds(0, SCH)], src[b], sip[b]).wait()

    def fire_gather(b):
        pltpu.async_copy(table_hbm.at[idx[b]], src[b], sg[b], add=True)

    def wait_g(b):
        pltpu.make_async_copy(pos_hbm.at[pl.ds(0, SCH)], src[b], sg[b]).wait()

    def transform(b):
        @pl.loop(0, SCH, unroll=4)
        def _row(i):
            ivec = jnp.full((16,), i, dtype=jnp.int32)
            for j in range(D // 16):
                v = src[b][i, pl.ds(16 * j, 16)]
                plsc.store_scatter(obuf[b], [dv[j], ivec], v)

    def fire_out(g, b):
        bidx = b0 + g // NST
        st = lax.rem(g, NST)
        for dtile in range(D // 8):
            pltpu.async_copy(
                obuf[b].at[pl.ds(8 * dtile, 8), pl.ds(0, SCH)],
                out_hbm.at[bidx, dtile, st, :, :],
                so[b],
            )

    def wait_out(b):
        for dtile in range(D // 8):
            pltpu.make_async_copy(
                obuf[b].at[pl.ds(8 * dtile, 8), pl.ds(0, SCH)],
                out_hbm.at[0, dtile, 0, :, :],
                so[b],
            ).wait()

    for b in range(NBUF):
        start(b, b)
    for b in range(NBUF):
        wait_ip(b)
        fire_gather(b)

    @pl.loop(0, NCHUNK, step=NBUF)
    def _go(go):
        for b in range(NBUF):
            g = go + b
            wait_g(b)

            @pl.when(g >= NBUF)
            def _(b=b):
                wait_out(b)

            transform(b)
            fire_out(g, b)

            @pl.when(g + NBUF < NCHUNK)
            def _(g=g, b=b):
                start(g + NBUF, b)

        for b in range(NBUF):

            @pl.when(go + NBUF + b < NCHUNK)
            def _(b=b):
                wait_ip(b)
                fire_gather(b)

    for b in range(NBUF):
        wait_out(b)


def kernel(tile_values, value_table, pos_table):
    out = _embed(tile_values.astype(jnp.int32), value_table, pos_table)
    return out.transpose(0, 2, 4, 1, 3).reshape(B, S, D)


# speedup vs baseline: 1.0050x; 1.0050x over previous
import functools

import jax
import jax.numpy as jnp
from jax import lax
from jax.experimental import pallas as pl
from jax.experimental.pallas import tpu as pltpu
from jax.experimental.pallas import tpu_sc as plsc

B = 1024
S = 1024
D = 64
NC, NS = 2, 16
NW = NC * NS
BPW = B // NW
SCH = 128
NST = S // SCH
NCHUNK = BPW * NST
NBUF = 4

_mesh = plsc.VectorSubcoreMesh(
    core_axis_name="c", subcore_axis_name="s", num_cores=NC, num_subcores=NS
)


@functools.partial(
    pl.kernel,
    out_type=jax.ShapeDtypeStruct((B, D // 8, S // 128, 8, 128), jnp.float32),
    mesh=_mesh,
    scratch_types=(
        [pltpu.VMEM((SCH,), jnp.int32) for _ in range(NBUF)]
        + [pltpu.VMEM((SCH, D), jnp.float32) for _ in range(NBUF)]
        + [pltpu.VMEM((D // 8, 8, SCH + 1), jnp.float32) for _ in range(NBUF)]
        + [pltpu.SemaphoreType.DMA for _ in range(3 * NBUF)]
    ),
    compiler_params=pltpu.CompilerParams(
        use_tc_tiling_on_sc=False,
        needs_layout_passes=False,
        disable_bounds_checks=True,
    ),
)
def _embed(tv_hbm, table_hbm, pos_hbm, out_hbm, *scratch):
    idx = scratch[:NBUF]
    src = scratch[NBUF:2 * NBUF]
    obuf = scratch[2 * NBUF:3 * NBUF]
    sip = scratch[3 * NBUF:4 * NBUF]
    sg = scratch[4 * NBUF:5 * NBUF]
    so = scratch[5 * NBUF:6 * NBUF]
    wid = lax.axis_index("s") * NC + lax.axis_index("c")
    b0 = wid * BPW

    iota = lax.iota(jnp.int32, 16)
    dr = lax.rem(iota, 8)
    dt = [lax.div(iota, 8) + 2 * j for j in range(D // 16)]

    def start(g, b):
        bidx = b0 + g // NST
        s0 = lax.rem(g, NST) * SCH
        pltpu.async_copy(tv_hbm.at[bidx, pl.ds(s0, SCH)], idx[b], sip[b])
        pltpu.async_copy(pos_hbm.at[pl.ds(s0, SCH)], src[b], sip[b])

    def wait_ip(b):
        pltpu.make_async_copy(tv_hbm.at[0, pl.ds(0, SCH)], idx[b], sip[b]).wait()
        pltpu.make_async_copy(pos_hbm.at[pl.ds(0, SCH)], src[b], sip[b]).wait()

    def fire_gather(b):
        pltpu.async_copy(table_hbm.at[idx[b]], src[b], sg[b], add=True)

    def wait_g(b):
        pltpu.make_async_copy(pos_hbm.at[pl.ds(0, SCH)], src[b], sg[b]).wait()

    def transform(b):
        @pl.loop(0, SCH, unroll=4)
        def _row(i):
            ivec = jnp.full((16,), i, dtype=jnp.int32)
            for j in range(D // 16):
                v = src[b][i, pl.ds(16 * j, 16)]
                plsc.store_scatter(obuf[b], [dt[j], dr, ivec], v)

    def fire_out(g, b):
        bidx = b0 + g // NST
        st = lax.rem(g, NST)
        pltpu.async_copy(
            obuf[b].at[:, :, pl.ds(0, SCH)], out_hbm.at[bidx, :, st, :, :], so[b]
        )

    def wait_out(b):
        pltpu.make_async_copy(
            obuf[b].at[:, :, pl.ds(0, SCH)], out_hbm.at[0, :, 0, :, :], so[b]
        ).wait()

    for b in range(NBUF):
        start(b, b)
    for b in range(NBUF):
        wait_ip(b)
        fire_gather(b)

    @pl.loop(0, NCHUNK, step=NBUF)
    def _go(go):
        for b in range(NBUF):
            g = go + b
            wait_g(b)

            @pl.when(g >= NBUF)
            def _(b=b):
                wait_out(b)

            transform(b)
            fire_out(g, b)

            @pl.when(g + NBUF < NCHUNK)
            def _(g=g, b=b):
                start(g + NBUF, b)

        for b in range(NBUF):

            @pl.when(go + NBUF + b < NCHUNK)
            def _(b=b):
                wait_ip(b)
                fire_gather(b)

    for b in range(NBUF):
        wait_out(b)


def kernel(tile_values, value_table, pos_table):
    out = _embed(tile_values.astype(jnp.int32), value_table, pos_table)
    return out.transpose(0, 2, 4, 1, 3).reshape(B, S, D)
